# baseline (speedup 1.0000x reference)
import functools

import jax
import jax.numpy as jnp
from jax.experimental import pallas as pl
from jax.experimental.pallas import tpu as pltpu

_B = 16
_F = 64
_N = 100000
_K = 4
_BN = 2048
_G = (_N + _BN - 1) // _BN

_INF = float("inf")


def _body(xa_ref, fp_ref, posT_ref, out_ref, rd_ref, rpx_ref, rpy_ref):
    i = pl.program_id(0)

    @pl.when(i == 0)
    def _init():
        rd_ref[...] = jnp.full((_B, _K), _INF, jnp.float32)
        rpx_ref[...] = jnp.zeros((_B, _K), jnp.float32)
        rpy_ref[...] = jnp.zeros((_B, _K), jnp.float32)

    fp = fp_ref[...]
    fsq = jnp.sum(fp * fp, axis=1, keepdims=True)
    ones = jnp.ones((_BN, 1), jnp.float32)
    b = jnp.concatenate([fp, fsq, ones], axis=1)
    d2 = jax.lax.dot_general(
        xa_ref[...], b, (((1,), (1,)), ((), ())),
        precision=jax.lax.Precision.HIGHEST,
        preferred_element_type=jnp.float32)

    col = jax.lax.broadcasted_iota(jnp.int32, (_B, _BN), 1)
    gcol = col + i * _BN
    d2 = jnp.where(gcol < _N, d2, _INF)

    pcol = jax.lax.broadcasted_iota(jnp.int32, (2, _BN), 1) + i * _BN
    posT = jnp.where(pcol < _N, posT_ref[...], 0.0)

    bds, bpxs, bpys = [], [], []
    for _ in range(_K):
        m = jnp.min(d2, axis=1, keepdims=True)
        hit = d2 == m
        sel = jnp.min(jnp.where(hit, col, _BN), axis=1, keepdims=True)
        oh = (col == sel).astype(jnp.float32)
        p = jax.lax.dot_general(
            oh, posT, (((1,), (1,)), ((), ())),
            precision=jax.lax.Precision.HIGHEST,
            preferred_element_type=jnp.float32)
        bds.append(m)
        bpxs.append(p[:, 0:1])
        bpys.append(p[:, 1:2])
        d2 = jnp.where(col == sel, _INF, d2)

    cd = jnp.concatenate([rd_ref[...]] + bds, axis=1)
    cpx = jnp.concatenate([rpx_ref[...]] + bpxs, axis=1)
    cpy = jnp.concatenate([rpy_ref[...]] + bpys, axis=1)
    cid = jax.lax.broadcasted_iota(jnp.int32, (_B, 2 * _K), 1)
    nds, npxs, npys = [], [], []
    for _ in range(_K):
        m = jnp.min(cd, axis=1, keepdims=True)
        sel = jnp.min(jnp.where(cd == m, cid, 2 * _K), axis=1, keepdims=True)
        oh = cid == sel
        npxs.append(jnp.sum(jnp.where(oh, cpx, 0.0), axis=1, keepdims=True))
        npys.append(jnp.sum(jnp.where(oh, cpy, 0.0), axis=1, keepdims=True))
        nds.append(m)
        cd = jnp.where(oh, _INF, cd)
    rd_ref[...] = jnp.concatenate(nds, axis=1)
    rpx_ref[...] = jnp.concatenate(npxs, axis=1)
    rpy_ref[...] = jnp.concatenate(npys, axis=1)

    @pl.when(i == _G - 1)
    def _finish():
        d = jnp.sqrt(rd_ref[...] + 1e-12)
        w = 1.0 / ((d + 1e-6) * (d + 1e-6))
        wn = w / (jnp.sum(w, axis=1, keepdims=True) + 1e-12)
        px = jnp.sum(rpx_ref[...] * wn, axis=1, keepdims=True)
        py = jnp.sum(rpy_ref[...] * wn, axis=1, keepdims=True)
        out_ref[...] = jnp.concatenate([px, py], axis=1)


@functools.partial(jax.jit, static_argnames=("interpret",))
def kernel(x, fingerprints, positions, interpret=False):
    xsq = jnp.sum(x * x, axis=1, keepdims=True)
    xa = jnp.concatenate(
        [-2.0 * x, jnp.ones((_B, 1), jnp.float32), xsq], axis=1)
    posT = positions.T
    return pl.pallas_call(
        _body,
        grid=(_G,),
        in_specs=[
            pl.BlockSpec((_B, _F + 2), lambda i: (0, 0)),
            pl.BlockSpec((_BN, _F), lambda i: (i, 0)),
            pl.BlockSpec((2, _BN), lambda i: (0, i)),
        ],
        out_specs=pl.BlockSpec((_B, 2), lambda i: (0, 0)),
        out_shape=jax.ShapeDtypeStruct((_B, 2), jnp.float32),
        scratch_shapes=[
            pltpu.VMEM((_B, _K), jnp.float32),
            pltpu.VMEM((_B, _K), jnp.float32),
            pltpu.VMEM((_B, _K), jnp.float32),
        ],
        compiler_params=pltpu.CompilerParams(
            dimension_semantics=("arbitrary",)),
        interpret=interpret,
    )(xa, fingerprints, posT)


# speedup vs baseline: 1.1756x; 1.1756x over previous
import functools

import jax
import jax.numpy as jnp
from jax import lax
from jax.experimental import pallas as pl
from jax.experimental.pallas import tpu as pltpu
from jax.experimental.pallas import tpu_sc as plsc

_B = 16
_F = 64
_N = 100000
_K = 4
_BN = 8192
_G = (_N + _BN - 1) // _BN

_INF = float("inf")


def _topk_body(xm2_ref, xsq_ref, fp_ref, dist_ref, idx_ref, rd_ref, ri_ref):
    i = pl.program_id(0)

    @pl.when(i == 0)
    def _init():
        rd_ref[...] = jnp.full((_B, _K), _INF, jnp.float32)
        ri_ref[...] = jnp.zeros((_B, _K), jnp.int32)

    fp = fp_ref[...]
    fp2 = fp * fp
    ones_row = jnp.ones((1, _F), jnp.float32)
    fsq = jax.lax.dot_general(
        ones_row, fp2, (((1,), (1,)), ((), ())),
        precision=jax.lax.Precision.HIGHEST,
        preferred_element_type=jnp.float32)
    d2 = jax.lax.dot_general(
        xm2_ref[...], fp, (((1,), (1,)), ((), ())),
        precision=jax.lax.Precision.HIGHEST,
        preferred_element_type=jnp.float32) + fsq

    col = jax.lax.broadcasted_iota(jnp.int32, (_B, _BN), 1)
    gcol = col + i * _BN
    d2 = jnp.where(gcol < _N, d2, _INF)

    bds, bis = [], []
    for _ in range(_K):
        m = jnp.min(d2, axis=1, keepdims=True)
        sel = jnp.min(jnp.where(d2 == m, col, _BN), axis=1, keepdims=True)
        bds.append(m)
        bis.append(sel + i * _BN)
        d2 = jnp.where(col == sel, _INF, d2)

    cd = jnp.concatenate([rd_ref[...]] + bds, axis=1)
    ci = jnp.concatenate([ri_ref[...]] + bis, axis=1)
    cid = jax.lax.broadcasted_iota(jnp.int32, (_B, 2 * _K), 1)
    nds, nis = [], []
    for _ in range(_K):
        m = jnp.min(cd, axis=1, keepdims=True)
        sel = jnp.min(jnp.where(cd == m, cid, 2 * _K), axis=1, keepdims=True)
        oh = cid == sel
        nis.append(jnp.sum(jnp.where(oh, ci, 0), axis=1, keepdims=True))
        nds.append(m)
        cd = jnp.where(oh, _INF, cd)
    rd_ref[...] = jnp.concatenate(nds, axis=1)
    ri_ref[...] = jnp.concatenate(nis, axis=1)

    @pl.when(i == _G - 1)
    def _finish():
        dist_ref[...] = jnp.sqrt(rd_ref[...] + xsq_ref[...] + 1e-12)
        idx_ref[...] = ri_ref[...]


def _topk_call(xm2, xsq, fingerprints):
    return pl.pallas_call(
        _topk_body,
        grid=(_G,),
        in_specs=[
            pl.BlockSpec((_B, _F), lambda i: (0, 0)),
            pl.BlockSpec((_B, 1), lambda i: (0, 0)),
            pl.BlockSpec((_BN, _F), lambda i: (i, 0)),
        ],
        out_specs=[
            pl.BlockSpec((_B, _K), lambda i: (0, 0)),
            pl.BlockSpec((_B, _K), lambda i: (0, 0)),
        ],
        out_shape=[
            jax.ShapeDtypeStruct((_B, _K), jnp.float32),
            jax.ShapeDtypeStruct((_B, _K), jnp.int32),
        ],
        scratch_shapes=[
            pltpu.VMEM((_B, _K), jnp.float32),
            pltpu.VMEM((_B, _K), jnp.int32),
        ],
        compiler_params=pltpu.CompilerParams(
            dimension_semantics=("arbitrary",)),
    )(xm2, xsq, fingerprints)


def _combine_body(dist_hbm, idx_hbm, posx_hbm, posy_hbm, out_hbm,
                  d_v, i_v, px_v, py_v, o_v, sem):
    cid = lax.axis_index("c")
    sid = lax.axis_index("s")

    @pl.when(jnp.logical_and(cid == 0, sid == 0))
    def _():
        pltpu.sync_copy(dist_hbm, d_v)
        pltpu.sync_copy(idx_hbm, i_v)
        pltpu.async_copy(posx_hbm.at[i_v], px_v, sem).wait()
        pltpu.async_copy(posy_hbm.at[i_v], py_v, sem).wait()
        ws, wpx, wpy = None, None, None
        for k in range(_K):
            d = d_v[k, :]
            w = 1.0 / ((d + 1e-6) * (d + 1e-6))
            px = px_v[pl.ds(k * _B, _B)]
            py = py_v[pl.ds(k * _B, _B)]
            ws = w if ws is None else ws + w
            wpx = w * px if wpx is None else wpx + w * px
            wpy = w * py if wpy is None else wpy + w * py
        inv = 1.0 / (ws + 1e-12)
        o_v[0, :] = wpx * inv
        o_v[1, :] = wpy * inv
        pltpu.sync_copy(o_v, out_hbm)


@functools.cache
def _build_combine():
    return functools.partial(
        pl.kernel,
        out_type=jax.ShapeDtypeStruct((2, _B), jnp.float32),
        mesh=plsc.VectorSubcoreMesh(core_axis_name="c", subcore_axis_name="s"),
        scratch_types=[
            pltpu.VMEM((_K, _B), jnp.float32),
            pltpu.VMEM((_K * _B,), jnp.int32),
            pltpu.VMEM((_K * _B,), jnp.float32),
            pltpu.VMEM((_K * _B,), jnp.float32),
            pltpu.VMEM((2, _B), jnp.float32),
            pltpu.SemaphoreType.DMA,
        ],
    )(_combine_body)


@jax.jit
def kernel(x, fingerprints, positions):
    xm2 = -2.0 * x
    xsq = jnp.sum(x * x, axis=1, keepdims=True)
    dist, idx = _topk_call(xm2, xsq, fingerprints)
    dist_km = dist.T
    idx_km = idx.T.reshape(_K * _B)
    posx = positions[:, 0]
    posy = positions[:, 1]
    out = _build_combine()(dist_km, idx_km, posx, posy)
    return out.T
